# baseline (speedup 1.0000x reference)
import functools

import jax
import jax.numpy as jnp
from jax import lax
from jax.experimental import pallas as pl
from jax.experimental.pallas import tpu as pltpu
from jax.experimental.pallas import tpu_sc as plsc

N = 10000
C = 256
K = 16
HID = 1024
EPS = 1e-5

BLK = 2000
NBLK = N // BLK
SBLK = 1000

NW = 32
RPW = 320
NPAD = NW * RPW
NB = 8
NBATCH = RPW // NB

_PREC = lax.Precision.DEFAULT


def _dot(a, b):
    return jnp.dot(a.astype(jnp.bfloat16), b.astype(jnp.bfloat16),
                   precision=_PREC, preferred_element_type=jnp.float32)


CP = C // 2


def _gelu(x):
    return 0.5 * x * (1.0 + lax.erf(x * (1.0 / jnp.sqrt(2.0).astype(jnp.float32))))


def _pack_bf16(x):
    lo = x[:, :CP].astype(jnp.bfloat16).astype(jnp.float32)
    hi = x[:, CP:].astype(jnp.bfloat16).astype(jnp.float32)
    loi = lax.shift_right_logical(lax.bitcast_convert_type(lo, jnp.int32), 16)
    hii = lax.bitcast_convert_type(hi, jnp.int32) & jnp.int32(-65536)
    return lax.bitcast_convert_type(hii | loi, jnp.float32)


def _i32(v):
    return lax.bitcast_convert_type(v, jnp.int32)


def _f32(v):
    return lax.bitcast_convert_type(v, jnp.float32)


def _shl16(w):
    return _f32(lax.shift_left(_i32(w), 16))


def _unpack_bf16(pp):
    pi = lax.bitcast_convert_type(pp, jnp.int32)
    lo = lax.bitcast_convert_type(lax.shift_left(pi, 16), jnp.float32)
    hi = lax.bitcast_convert_type(pi & jnp.int32(-65536), jnp.float32)
    return jnp.concatenate([lo, hi], axis=1)


def _bn_from_stats(v, stats_ref, g_ref, b_ref):
    mu = stats_ref[0:1, :] * (1.0 / N)
    var = stats_ref[1:2, :] * (1.0 / N) - mu * mu
    inv = lax.rsqrt(var + EPS) * g_ref[...]
    return (v - mu) * inv + b_ref[...]


def _acc_stats(i, y, stats_ref):
    @pl.when(i == 0)
    def _():
        stats_ref[...] = jnp.zeros_like(stats_ref)

    stats_ref[0:1, :] += jnp.sum(y, axis=0, keepdims=True)
    stats_ref[1:2, :] += jnp.sum(y * y, axis=0, keepdims=True)


def _mlp_body(f_ref, w1_ref, b1_ref, w2_ref, y_ref, stats_ref):
    i = pl.program_id(0)
    h = _gelu(_dot(f_ref[...], w1_ref[...]) + b1_ref[...])
    y = _dot(h, w2_ref[...])
    y_ref[...] = y
    _acc_stats(i, y, stats_ref)


def _packed_stats(i, p_ref, stats_v):
    @pl.when(i == 0)
    def _():
        def chunk(t, acc):
            yu = _unpack_bf16(p_ref[pl.ds(t * SBLK, SBLK), :])
            return (acc[0] + jnp.sum(yu, axis=0, keepdims=True),
                    acc[1] + jnp.sum(yu * yu, axis=0, keepdims=True))

        z = jnp.zeros((1, C), jnp.float32)
        s0, s1 = lax.fori_loop(0, N // SBLK, chunk, (z, z))
        stats_v[0:1, :] = s0
        stats_v[1:2, :] = s1


def _bnmm_body(f_ref, y_ref, stats_ref, g_ref, b_ref, w_ref,
               fn_ref, x_ref):
    fn = f_ref[...] + _bn_from_stats(y_ref[...], stats_ref, g_ref, b_ref)
    fn_ref[...] = fn
    x_ref[...] = _pack_bf16(_dot(fn, w_ref[...]))


def _bnmm_packed_body(f_ref, p_ref, g_ref, b_ref, w_ref,
                      fn_ref, x_ref, stats_v):
    i = pl.program_id(0)
    _packed_stats(i, p_ref, stats_v)
    p = _unpack_bf16(p_ref[pl.ds(i * BLK, BLK), :])
    fn = f_ref[...] + _bn_from_stats(p, stats_v, g_ref, b_ref)
    fn_ref[...] = fn
    x_ref[...] = _pack_bf16(_dot(fn, w_ref[...]))


def _bnmlp_body(f_ref, p_ref, g_ref, b_ref, w1_ref, b1_ref,
                w2_ref, fn_ref, y_ref, ostats_ref, stats_v):
    i = pl.program_id(0)
    _packed_stats(i, p_ref, stats_v)
    p = _unpack_bf16(p_ref[pl.ds(i * BLK, BLK), :])
    fn = f_ref[...] + _bn_from_stats(p, stats_v, g_ref, b_ref)
    fn_ref[...] = fn
    h = _gelu(_dot(fn, w1_ref[...]) + b1_ref[...])
    y = _dot(h, w2_ref[...])
    y_ref[...] = y
    _acc_stats(i, y, ostats_ref)


def _bnadd_body(f_ref, y_ref, stats_ref, g_ref, b_ref, out_ref):
    out_ref[...] = f_ref[...] + _bn_from_stats(y_ref[...], stats_ref,
                                               g_ref, b_ref)


def _row_spec(rows=BLK, cols=C):
    return pl.BlockSpec((rows, cols), lambda i: (i, 0))


def _full_spec(shape):
    return pl.BlockSpec(shape, lambda i: tuple(0 for _ in shape))


_SEQ = pltpu.CompilerParams(dimension_semantics=("arbitrary",))


def _mlp_call(f, w1, b1, w2):
    return pl.pallas_call(
        _mlp_body,
        grid=(NBLK,),
        in_specs=[_row_spec(), _full_spec((C, HID)), _full_spec((1, HID)),
                  _full_spec((HID, C))],
        out_specs=[_row_spec(), _full_spec((2, C))],
        out_shape=[jax.ShapeDtypeStruct((N, C), jnp.float32),
                   jax.ShapeDtypeStruct((2, C), jnp.float32)],
        compiler_params=_SEQ,
    )(f, w1, b1, w2)


def _bnmm_call(f, y, stats, g, b, w):
    return pl.pallas_call(
        _bnmm_body,
        grid=(NBLK,),
        in_specs=[_row_spec(), _row_spec(), _full_spec((2, C)),
                  _full_spec((1, C)), _full_spec((1, C)), _full_spec((C, C))],
        out_specs=[_row_spec(), _row_spec(cols=CP)],
        out_shape=[jax.ShapeDtypeStruct((N, C), jnp.float32),
                   jax.ShapeDtypeStruct((N, CP), jnp.float32)],
        compiler_params=_SEQ,
    )(f, y, stats, g, b, w)


def _bnmm_packed_call(f, p, g, b, w):
    return pl.pallas_call(
        _bnmm_packed_body,
        grid=(NBLK,),
        in_specs=[_row_spec(), _full_spec((N, CP)),
                  _full_spec((1, C)), _full_spec((1, C)), _full_spec((C, C))],
        out_specs=[_row_spec(), _row_spec(cols=CP)],
        out_shape=[jax.ShapeDtypeStruct((N, C), jnp.float32),
                   jax.ShapeDtypeStruct((N, CP), jnp.float32)],
        scratch_shapes=[pltpu.VMEM((2, C), jnp.float32)],
        compiler_params=_SEQ,
    )(f, p, g, b, w)


def _bnmlp_call(f, p, g, b, w1, b1, w2):
    return pl.pallas_call(
        _bnmlp_body,
        grid=(NBLK,),
        in_specs=[_row_spec(), _full_spec((N, CP)),
                  _full_spec((1, C)), _full_spec((1, C)),
                  _full_spec((C, HID)), _full_spec((1, HID)),
                  _full_spec((HID, C))],
        out_specs=[_row_spec(), _row_spec(), _full_spec((2, C))],
        out_shape=[jax.ShapeDtypeStruct((N, C), jnp.float32),
                   jax.ShapeDtypeStruct((N, C), jnp.float32),
                   jax.ShapeDtypeStruct((2, C), jnp.float32)],
        scratch_shapes=[pltpu.VMEM((2, C), jnp.float32)],
        compiler_params=_SEQ,
    )(f, p, g, b, w1, b1, w2)


def _bnadd_call(f, y, stats, g, b):
    return pl.pallas_call(
        _bnadd_body,
        grid=(NBLK,),
        in_specs=[_row_spec(), _row_spec(), _full_spec((2, C)),
                  _full_spec((1, C)), _full_spec((1, C))],
        out_specs=[_row_spec()],
        out_shape=[jax.ShapeDtypeStruct((N, C), jnp.float32)],
        compiler_params=_SEQ,
    )(f, y, stats, g, b)[0]


def _sc_gather_max(x_hbm, gidx_hbm, out_hbm, x_sh, idx_v, rows_v, out_v,
                   gsem0, gsem1, osem0, osem1):
    sid = lax.axis_index("s")
    wid = sid * 2 + lax.axis_index("c")
    base = wid * RPW
    gsems = [gsem0, gsem1]
    osems = [osem0, osem1]

    @pl.when(sid < 15)
    def _():
        pltpu.sync_copy(x_hbm.at[pl.ds(sid * 624, 624)],
                        x_sh.at[pl.ds(sid * 624, 624)])

    @pl.when(sid == 15)
    def _():
        pltpu.sync_copy(x_hbm.at[pl.ds(15 * 624, N - 15 * 624)],
                        x_sh.at[pl.ds(15 * 624, N - 15 * 624)])

    nbatch = jnp.where(wid == NW - 1, (N - (NW - 1) * RPW) // NB, NBATCH)

    @pl.when(wid < NW - 1)
    def _():
        pltpu.sync_copy(gidx_hbm.at[pl.ds(base * K, RPW * K)], idx_v)

    @pl.when(wid == NW - 1)
    def _():
        tail = (N - (NW - 1) * RPW) * K
        pltpu.sync_copy(gidx_hbm.at[pl.ds((NW - 1) * RPW * K, tail)],
                        idx_v.at[pl.ds(0, tail)])

    plsc.subcore_barrier()

    def gcopy(b, s):
        return pltpu.make_async_copy(
            x_sh.at[idx_v.at[pl.ds(b * (NB * K), NB * K)]],
            rows_v.at[s], gsems[s])

    def ocopy(b, s):
        return pltpu.make_async_copy(
            out_v.at[s], out_hbm.at[pl.ds(base + b * NB, NB)], osems[s])

    def half(b, s):
        @pl.when(b + 1 < nbatch)
        def _():
            gcopy(b + 1, 1 - s).start()

        gcopy(b, s).wait()

        @pl.when(b >= 2)
        def _():
            ocopy(b - 2, s).wait()

        rv = rows_v.at[s]
        ov = out_v.at[s]

        def node_body(j, c):
            r0 = j * K
            for g in range(CP // 16):
                sl = pl.ds(g * 16, 16)
                w0 = rv[r0, sl]
                acc_hi = w0
                acc_lo = _shl16(w0)
                for r in range(1, K):
                    w = rv[r0 + r, sl]
                    acc_hi = jnp.maximum(acc_hi, w)
                    acc_lo = jnp.maximum(acc_lo, _shl16(w))
                hi_bits = _i32(acc_hi) & jnp.int32(-65536)
                lo_bits = lax.shift_right_logical(_i32(acc_lo), 16)
                ov[j, sl] = _f32(hi_bits | lo_bits)
            return c

        lax.fori_loop(0, NB, node_body, 0)
        ocopy(b, s).start()

    gcopy(0, 0).start()

    def outer(t, carry):
        half(t * 2, 0)
        half(t * 2 + 1, 1)
        return carry

    lax.fori_loop(0, nbatch // 2, outer, 0)
    ocopy(nbatch - 2, 0).wait()
    ocopy(nbatch - 1, 1).wait()


def _sc_pool_call(x, gidx_flat):
    mesh = plsc.VectorSubcoreMesh(core_axis_name="c", subcore_axis_name="s")
    kfn = functools.partial(
        pl.kernel,
        mesh=mesh,
        out_type=jax.ShapeDtypeStruct((N, CP), jnp.float32),
        scratch_types=[
            pltpu.VMEM_SHARED((N, CP), jnp.float32),
            pltpu.VMEM((RPW * K,), jnp.int32),
            pltpu.VMEM((2, NB * K, CP), jnp.float32),
            pltpu.VMEM((2, NB, CP), jnp.float32),
            pltpu.SemaphoreType.DMA,
            pltpu.SemaphoreType.DMA,
            pltpu.SemaphoreType.DMA,
            pltpu.SemaphoreType.DMA,
        ],
    )(_sc_gather_max)
    return kfn(x, gidx_flat)


def kernel(f, group_idx, params):
    mlp0 = params["mlp0"]
    blocks = params["blocks"]
    mlps = params["mlps"]

    gidx = group_idx.astype(jnp.int32).reshape(-1)

    def r1(a):
        return a.reshape(1, -1)

    y, s = _mlp_call(f, mlp0["W1"], r1(mlp0["b1"]), mlp0["W2"])
    cur_g, cur_b = r1(mlp0["g"]), r1(mlp0["b"])
    fcur = f
    packed = False

    for i in range(4):
        if packed:
            fcur, x = _bnmm_packed_call(fcur, y, cur_g, cur_b,
                                        blocks[i]["Wproj"])
        else:
            fcur, x = _bnmm_call(fcur, y, s, cur_g, cur_b,
                                 blocks[i]["Wproj"])
        y = _sc_pool_call(x, gidx)
        packed = True
        cur_g, cur_b = r1(blocks[i]["g"]), r1(blocks[i]["b"])
        if i % 2 == 1:
            m = mlps[i // 2]
            fcur, y, s = _bnmlp_call(fcur, y, cur_g, cur_b,
                                     m["W1"], r1(m["b1"]), m["W2"])
            packed = False
            cur_g, cur_b = r1(m["g"]), r1(m["b"])

    return _bnadd_call(fcur, y, s, cur_g, cur_b)


# speedup vs baseline: 1.0013x; 1.0013x over previous
import functools

import jax
import jax.numpy as jnp
from jax import lax
from jax.experimental import pallas as pl
from jax.experimental.pallas import tpu as pltpu
from jax.experimental.pallas import tpu_sc as plsc

N = 10000
C = 256
K = 16
HID = 1024
EPS = 1e-5

BLK = 2000
NBLK = N // BLK
SBLK = 1000

NW = 32
RPW = 320
NPAD = NW * RPW
NB = 8
NBATCH = RPW // NB

_PREC = lax.Precision.DEFAULT


def _dot(a, b):
    return jnp.dot(a.astype(jnp.bfloat16), b.astype(jnp.bfloat16),
                   precision=_PREC, preferred_element_type=jnp.float32)


CP = C // 2


def _gelu(x):
    return 0.5 * x * (1.0 + lax.erf(x * (1.0 / jnp.sqrt(2.0).astype(jnp.float32))))


def _pack_bf16(x):
    lo = x[:, :CP].astype(jnp.bfloat16).astype(jnp.float32)
    hi = x[:, CP:].astype(jnp.bfloat16).astype(jnp.float32)
    loi = lax.shift_right_logical(lax.bitcast_convert_type(lo, jnp.int32), 16)
    hii = lax.bitcast_convert_type(hi, jnp.int32) & jnp.int32(-65536)
    return lax.bitcast_convert_type(hii | loi, jnp.float32)


def _i32(v):
    return lax.bitcast_convert_type(v, jnp.int32)


def _f32(v):
    return lax.bitcast_convert_type(v, jnp.float32)


def _shl16(w):
    return _f32(lax.shift_left(_i32(w), 16))


def _unpack_bf16(pp):
    pi = lax.bitcast_convert_type(pp, jnp.int32)
    lo = lax.bitcast_convert_type(lax.shift_left(pi, 16), jnp.float32)
    hi = lax.bitcast_convert_type(pi & jnp.int32(-65536), jnp.float32)
    return jnp.concatenate([lo, hi], axis=1)


def _bn_from_stats(v, stats_ref, g_ref, b_ref):
    mu = stats_ref[0:1, :] * (1.0 / N)
    var = stats_ref[1:2, :] * (1.0 / N) - mu * mu
    inv = lax.rsqrt(var + EPS) * g_ref[...]
    return (v - mu) * inv + b_ref[...]


def _acc_stats(i, y, stats_ref):
    @pl.when(i == 0)
    def _():
        stats_ref[...] = jnp.zeros_like(stats_ref)

    stats_ref[0:1, :] += jnp.sum(y, axis=0, keepdims=True)
    stats_ref[1:2, :] += jnp.sum(y * y, axis=0, keepdims=True)


def _mlp_body(f_ref, w1_ref, b1_ref, w2_ref, y_ref, stats_ref):
    i = pl.program_id(0)
    h = _gelu(_dot(f_ref[...], w1_ref[...]) + b1_ref[...])
    y = _dot(h, w2_ref[...])
    y_ref[...] = y
    _acc_stats(i, y, stats_ref)


def _packed_stats(i, p_ref, stats_v):
    @pl.when(i == 0)
    def _():
        def chunk(t, acc):
            yu = _unpack_bf16(p_ref[pl.ds(t * SBLK, SBLK), :])
            return (acc[0] + jnp.sum(yu, axis=0, keepdims=True),
                    acc[1] + jnp.sum(yu * yu, axis=0, keepdims=True))

        z = jnp.zeros((1, C), jnp.float32)
        s0, s1 = lax.fori_loop(0, N // SBLK, chunk, (z, z))
        stats_v[0:1, :] = s0
        stats_v[1:2, :] = s1


def _bnmm_body(f_ref, y_ref, stats_ref, g_ref, b_ref, w_ref,
               fn_ref, x_ref):
    fn = f_ref[...] + _bn_from_stats(y_ref[...], stats_ref, g_ref, b_ref)
    fn_ref[...] = fn
    x_ref[...] = _pack_bf16(_dot(fn, w_ref[...]))


def _bnmm_packed_body(f_ref, p_ref, g_ref, b_ref, w_ref,
                      fn_ref, x_ref, stats_v):
    i = pl.program_id(0)
    _packed_stats(i, p_ref, stats_v)
    p = _unpack_bf16(p_ref[pl.ds(i * BLK, BLK), :])
    fn = f_ref[...] + _bn_from_stats(p, stats_v, g_ref, b_ref)
    fn_ref[...] = fn
    x_ref[...] = _pack_bf16(_dot(fn, w_ref[...]))


def _bnmlp_body(f_ref, p_ref, g_ref, b_ref, w1_ref, b1_ref,
                w2_ref, fn_ref, y_ref, ostats_ref, stats_v):
    i = pl.program_id(0)
    _packed_stats(i, p_ref, stats_v)
    p = _unpack_bf16(p_ref[pl.ds(i * BLK, BLK), :])
    fn = f_ref[...] + _bn_from_stats(p, stats_v, g_ref, b_ref)
    fn_ref[...] = fn
    h = _gelu(_dot(fn, w1_ref[...]) + b1_ref[...])
    y = _dot(h, w2_ref[...])
    y_ref[...] = y
    _acc_stats(i, y, ostats_ref)


def _bnadd_body(f_ref, y_ref, stats_ref, g_ref, b_ref, out_ref):
    out_ref[...] = f_ref[...] + _bn_from_stats(y_ref[...], stats_ref,
                                               g_ref, b_ref)


def _row_spec(rows=BLK, cols=C):
    return pl.BlockSpec((rows, cols), lambda i: (i, 0))


def _full_spec(shape):
    return pl.BlockSpec(shape, lambda i: tuple(0 for _ in shape))


_SEQ = pltpu.CompilerParams(dimension_semantics=("arbitrary",))


def _mlp_call(f, w1, b1, w2):
    return pl.pallas_call(
        _mlp_body,
        grid=(NBLK,),
        in_specs=[_row_spec(), _full_spec((C, HID)), _full_spec((1, HID)),
                  _full_spec((HID, C))],
        out_specs=[_row_spec(), _full_spec((2, C))],
        out_shape=[jax.ShapeDtypeStruct((N, C), jnp.float32),
                   jax.ShapeDtypeStruct((2, C), jnp.float32)],
        compiler_params=_SEQ,
    )(f, w1, b1, w2)


def _bnmm_call(f, y, stats, g, b, w):
    return pl.pallas_call(
        _bnmm_body,
        grid=(NBLK,),
        in_specs=[_row_spec(), _row_spec(), _full_spec((2, C)),
                  _full_spec((1, C)), _full_spec((1, C)), _full_spec((C, C))],
        out_specs=[_row_spec(), _row_spec(cols=CP)],
        out_shape=[jax.ShapeDtypeStruct((N, C), jnp.float32),
                   jax.ShapeDtypeStruct((N, CP), jnp.float32)],
        compiler_params=_SEQ,
    )(f, y, stats, g, b, w)


def _bnmm_packed_call(f, p, g, b, w):
    return pl.pallas_call(
        _bnmm_packed_body,
        grid=(NBLK,),
        in_specs=[_row_spec(), _full_spec((N, CP)),
                  _full_spec((1, C)), _full_spec((1, C)), _full_spec((C, C))],
        out_specs=[_row_spec(), _row_spec(cols=CP)],
        out_shape=[jax.ShapeDtypeStruct((N, C), jnp.float32),
                   jax.ShapeDtypeStruct((N, CP), jnp.float32)],
        scratch_shapes=[pltpu.VMEM((2, C), jnp.float32)],
        compiler_params=_SEQ,
    )(f, p, g, b, w)


def _bnmlp_call(f, p, g, b, w1, b1, w2):
    return pl.pallas_call(
        _bnmlp_body,
        grid=(NBLK,),
        in_specs=[_row_spec(), _full_spec((N, CP)),
                  _full_spec((1, C)), _full_spec((1, C)),
                  _full_spec((C, HID)), _full_spec((1, HID)),
                  _full_spec((HID, C))],
        out_specs=[_row_spec(), _row_spec(), _full_spec((2, C))],
        out_shape=[jax.ShapeDtypeStruct((N, C), jnp.float32),
                   jax.ShapeDtypeStruct((N, C), jnp.float32),
                   jax.ShapeDtypeStruct((2, C), jnp.float32)],
        scratch_shapes=[pltpu.VMEM((2, C), jnp.float32)],
        compiler_params=_SEQ,
    )(f, p, g, b, w1, b1, w2)


def _bnadd_call(f, y, stats, g, b):
    return pl.pallas_call(
        _bnadd_body,
        grid=(NBLK,),
        in_specs=[_row_spec(), _row_spec(), _full_spec((2, C)),
                  _full_spec((1, C)), _full_spec((1, C))],
        out_specs=[_row_spec()],
        out_shape=[jax.ShapeDtypeStruct((N, C), jnp.float32)],
        compiler_params=_SEQ,
    )(f, y, stats, g, b)[0]


def _sc_gather_max(x_hbm, gidx_hbm, out_hbm, x_sh, idx_v, rows_v, out_v,
                   gsem0, gsem1, osem0, osem1):
    sid = lax.axis_index("s")
    wid = sid * 2 + lax.axis_index("c")
    base = wid * RPW
    gsems = [gsem0, gsem1]
    osems = [osem0, osem1]

    @pl.when(sid < 15)
    def _():
        pltpu.sync_copy(x_hbm.at[pl.ds(sid * 624, 624)],
                        x_sh.at[pl.ds(sid * 624, 624)])

    @pl.when(sid == 15)
    def _():
        pltpu.sync_copy(x_hbm.at[pl.ds(15 * 624, N - 15 * 624)],
                        x_sh.at[pl.ds(15 * 624, N - 15 * 624)])

    nbatch = jnp.where(wid == NW - 1, (N - (NW - 1) * RPW) // NB, NBATCH)

    @pl.when(wid < NW - 1)
    def _():
        pltpu.sync_copy(gidx_hbm.at[pl.ds(base * K, RPW * K)], idx_v)

    @pl.when(wid == NW - 1)
    def _():
        tail = (N - (NW - 1) * RPW) * K
        pltpu.sync_copy(gidx_hbm.at[pl.ds((NW - 1) * RPW * K, tail)],
                        idx_v.at[pl.ds(0, tail)])

    plsc.subcore_barrier()

    def gcopy(b, s):
        return pltpu.make_async_copy(
            x_sh.at[idx_v.at[pl.ds(b * (NB * K), NB * K)]],
            rows_v.at[s], gsems[s])

    def ocopy(b, s):
        return pltpu.make_async_copy(
            out_v.at[s], out_hbm.at[pl.ds(base + b * NB, NB)], osems[s])

    def half(b, s):
        @pl.when(b + 1 < nbatch)
        def _():
            gcopy(b + 1, 1 - s).start()

        gcopy(b, s).wait()

        @pl.when(b >= 2)
        def _():
            ocopy(b - 2, s).wait()

        rv = rows_v.at[s]
        ov = out_v.at[s]

        def node_body(j, c):
            r0 = j * K
            for g in range(CP // 16):
                sl = pl.ds(g * 16, 16)
                w0 = rv[r0, sl]
                acc_hi = w0
                acc_lo = _shl16(w0)
                for r in range(1, K):
                    w = rv[r0 + r, sl]
                    acc_hi = jnp.maximum(acc_hi, w)
                    acc_lo = jnp.maximum(acc_lo, _shl16(w))
                hi_bits = _i32(acc_hi) & jnp.int32(-65536)
                lo_bits = lax.shift_right_logical(_i32(acc_lo), 16)
                ov[j, sl] = _f32(hi_bits | lo_bits)
            return c

        lax.fori_loop(0, NB, node_body, 0, unroll=2)
        ocopy(b, s).start()

    gcopy(0, 0).start()

    def outer(t, carry):
        half(t * 2, 0)
        half(t * 2 + 1, 1)
        return carry

    lax.fori_loop(0, nbatch // 2, outer, 0)
    ocopy(nbatch - 2, 0).wait()
    ocopy(nbatch - 1, 1).wait()


def _sc_pool_call(x, gidx_flat):
    mesh = plsc.VectorSubcoreMesh(core_axis_name="c", subcore_axis_name="s")
    kfn = functools.partial(
        pl.kernel,
        mesh=mesh,
        out_type=jax.ShapeDtypeStruct((N, CP), jnp.float32),
        scratch_types=[
            pltpu.VMEM_SHARED((N, CP), jnp.float32),
            pltpu.VMEM((RPW * K,), jnp.int32),
            pltpu.VMEM((2, NB * K, CP), jnp.float32),
            pltpu.VMEM((2, NB, CP), jnp.float32),
            pltpu.SemaphoreType.DMA,
            pltpu.SemaphoreType.DMA,
            pltpu.SemaphoreType.DMA,
            pltpu.SemaphoreType.DMA,
        ],
    )(_sc_gather_max)
    return kfn(x, gidx_flat)


def kernel(f, group_idx, params):
    mlp0 = params["mlp0"]
    blocks = params["blocks"]
    mlps = params["mlps"]

    gidx = group_idx.astype(jnp.int32).reshape(-1)

    def r1(a):
        return a.reshape(1, -1)

    y, s = _mlp_call(f, mlp0["W1"], r1(mlp0["b1"]), mlp0["W2"])
    cur_g, cur_b = r1(mlp0["g"]), r1(mlp0["b"])
    fcur = f
    packed = False

    for i in range(4):
        if packed:
            fcur, x = _bnmm_packed_call(fcur, y, cur_g, cur_b,
                                        blocks[i]["Wproj"])
        else:
            fcur, x = _bnmm_call(fcur, y, s, cur_g, cur_b,
                                 blocks[i]["Wproj"])
        y = _sc_pool_call(x, gidx)
        packed = True
        cur_g, cur_b = r1(blocks[i]["g"]), r1(blocks[i]["b"])
        if i % 2 == 1:
            m = mlps[i // 2]
            fcur, y, s = _bnmlp_call(fcur, y, cur_g, cur_b,
                                     m["W1"], r1(m["b1"]), m["W2"])
            packed = False
            cur_g, cur_b = r1(m["g"]), r1(m["b"])

    return _bnadd_call(fcur, y, s, cur_g, cur_b)


# speedup vs baseline: 1.0033x; 1.0019x over previous
import functools

import jax
import jax.numpy as jnp
from jax import lax
from jax.experimental import pallas as pl
from jax.experimental.pallas import tpu as pltpu
from jax.experimental.pallas import tpu_sc as plsc

N = 10000
C = 256
K = 16
HID = 1024
EPS = 1e-5

BLK = 2000
NBLK = N // BLK
SBLK = 1000

NW = 32
RPW = 320
NB = 8
NBATCH = RPW // NB

_PREC = lax.Precision.DEFAULT


def _dot(a, b):
    return jnp.dot(a.astype(jnp.bfloat16), b.astype(jnp.bfloat16),
                   precision=_PREC, preferred_element_type=jnp.float32)


CP = C // 2


def _gelu(x):
    return 0.5 * x * (1.0 + lax.erf(x * (1.0 / jnp.sqrt(2.0).astype(jnp.float32))))


def _pack_bf16(x):
    lo = x[:, :CP].astype(jnp.bfloat16).astype(jnp.float32)
    hi = x[:, CP:].astype(jnp.bfloat16).astype(jnp.float32)
    loi = lax.shift_right_logical(lax.bitcast_convert_type(lo, jnp.int32), 16)
    hii = lax.bitcast_convert_type(hi, jnp.int32) & jnp.int32(-65536)
    return lax.bitcast_convert_type(hii | loi, jnp.float32)


def _i32(v):
    return lax.bitcast_convert_type(v, jnp.int32)


def _f32(v):
    return lax.bitcast_convert_type(v, jnp.float32)


def _shl16(w):
    return _f32(lax.shift_left(_i32(w), 16))


def _unpack_bf16(pp):
    pi = lax.bitcast_convert_type(pp, jnp.int32)
    lo = lax.bitcast_convert_type(lax.shift_left(pi, 16), jnp.float32)
    hi = lax.bitcast_convert_type(pi & jnp.int32(-65536), jnp.float32)
    return jnp.concatenate([lo, hi], axis=1)


def _bn_from_stats(v, stats_ref, g_ref, b_ref):
    mu = stats_ref[0:1, :] * (1.0 / N)
    var = stats_ref[1:2, :] * (1.0 / N) - mu * mu
    inv = lax.rsqrt(var + EPS) * g_ref[...]
    return (v - mu) * inv + b_ref[...]


def _acc_stats(i, y, stats_ref):
    @pl.when(i == 0)
    def _():
        stats_ref[...] = jnp.zeros_like(stats_ref)

    stats_ref[0:1, :] += jnp.sum(y, axis=0, keepdims=True)
    stats_ref[1:2, :] += jnp.sum(y * y, axis=0, keepdims=True)


def _mlp_body(f_ref, w1_ref, b1_ref, w2_ref, y_ref, stats_ref):
    i = pl.program_id(0)
    h = _gelu(_dot(f_ref[...], w1_ref[...]) + b1_ref[...])
    y = _dot(h, w2_ref[...])
    y_ref[...] = y
    _acc_stats(i, y, stats_ref)


def _packed_stats(i, p_ref, stats_v):
    @pl.when(i == 0)
    def _():
        def chunk(t, acc):
            yu = _unpack_bf16(p_ref[pl.ds(t * SBLK, SBLK), :])
            return (acc[0] + jnp.sum(yu, axis=0, keepdims=True),
                    acc[1] + jnp.sum(yu * yu, axis=0, keepdims=True))

        z = jnp.zeros((1, C), jnp.float32)
        s0, s1 = lax.fori_loop(0, N // SBLK, chunk, (z, z))
        stats_v[0:1, :] = s0
        stats_v[1:2, :] = s1


def _bnmm_body(f_ref, y_ref, stats_ref, g_ref, b_ref, w_ref,
               fn_ref, x_ref):
    fn = f_ref[...] + _bn_from_stats(y_ref[...], stats_ref, g_ref, b_ref)
    fn_ref[...] = fn
    x_ref[...] = _pack_bf16(_dot(fn, w_ref[...]))


def _bnmm_packed_body(f_ref, p_ref, g_ref, b_ref, w_ref,
                      fn_ref, x_ref, stats_v):
    i = pl.program_id(0)
    _packed_stats(i, p_ref, stats_v)
    p = _unpack_bf16(p_ref[pl.ds(i * BLK, BLK), :])
    fn = f_ref[...] + _bn_from_stats(p, stats_v, g_ref, b_ref)
    fn_ref[...] = fn
    x_ref[...] = _pack_bf16(_dot(fn, w_ref[...]))


def _bnmlp_body(f_ref, p_ref, g_ref, b_ref, w1_ref, b1_ref,
                w2_ref, fn_ref, y_ref, ostats_ref, stats_v):
    i = pl.program_id(0)
    _packed_stats(i, p_ref, stats_v)
    p = _unpack_bf16(p_ref[pl.ds(i * BLK, BLK), :])
    fn = f_ref[...] + _bn_from_stats(p, stats_v, g_ref, b_ref)
    fn_ref[...] = fn
    h = _gelu(_dot(fn, w1_ref[...]) + b1_ref[...])
    y = _dot(h, w2_ref[...])
    y_ref[...] = y
    _acc_stats(i, y, ostats_ref)


def _bnadd_body(f_ref, y_ref, stats_ref, g_ref, b_ref, out_ref):
    out_ref[...] = f_ref[...] + _bn_from_stats(y_ref[...], stats_ref,
                                               g_ref, b_ref)


def _row_spec(rows=BLK, cols=C):
    return pl.BlockSpec((rows, cols), lambda i: (i, 0))


def _full_spec(shape):
    return pl.BlockSpec(shape, lambda i: tuple(0 for _ in shape))


_SEQ = pltpu.CompilerParams(dimension_semantics=("arbitrary",))


def _mlp_call(f, w1, b1, w2):
    return pl.pallas_call(
        _mlp_body,
        grid=(NBLK,),
        in_specs=[_row_spec(), _full_spec((C, HID)), _full_spec((1, HID)),
                  _full_spec((HID, C))],
        out_specs=[_row_spec(), _full_spec((2, C))],
        out_shape=[jax.ShapeDtypeStruct((N, C), jnp.float32),
                   jax.ShapeDtypeStruct((2, C), jnp.float32)],
        compiler_params=_SEQ,
    )(f, w1, b1, w2)


def _bnmm_call(f, y, stats, g, b, w):
    return pl.pallas_call(
        _bnmm_body,
        grid=(NBLK,),
        in_specs=[_row_spec(), _row_spec(), _full_spec((2, C)),
                  _full_spec((1, C)), _full_spec((1, C)), _full_spec((C, C))],
        out_specs=[_row_spec(), _row_spec(cols=CP)],
        out_shape=[jax.ShapeDtypeStruct((N, C), jnp.float32),
                   jax.ShapeDtypeStruct((N, CP), jnp.float32)],
        compiler_params=_SEQ,
    )(f, y, stats, g, b, w)


def _bnmm_packed_call(f, p, g, b, w):
    return pl.pallas_call(
        _bnmm_packed_body,
        grid=(NBLK,),
        in_specs=[_row_spec(), _full_spec((N, CP)),
                  _full_spec((1, C)), _full_spec((1, C)), _full_spec((C, C))],
        out_specs=[_row_spec(), _row_spec(cols=CP)],
        out_shape=[jax.ShapeDtypeStruct((N, C), jnp.float32),
                   jax.ShapeDtypeStruct((N, CP), jnp.float32)],
        scratch_shapes=[pltpu.VMEM((2, C), jnp.float32)],
        compiler_params=_SEQ,
    )(f, p, g, b, w)


def _bnmlp_call(f, p, g, b, w1, b1, w2):
    return pl.pallas_call(
        _bnmlp_body,
        grid=(NBLK,),
        in_specs=[_row_spec(), _full_spec((N, CP)),
                  _full_spec((1, C)), _full_spec((1, C)),
                  _full_spec((C, HID)), _full_spec((1, HID)),
                  _full_spec((HID, C))],
        out_specs=[_row_spec(), _row_spec(), _full_spec((2, C))],
        out_shape=[jax.ShapeDtypeStruct((N, C), jnp.float32),
                   jax.ShapeDtypeStruct((N, C), jnp.float32),
                   jax.ShapeDtypeStruct((2, C), jnp.float32)],
        scratch_shapes=[pltpu.VMEM((2, C), jnp.float32)],
        compiler_params=_SEQ,
    )(f, p, g, b, w1, b1, w2)


def _bnadd_call(f, y, stats, g, b):
    return pl.pallas_call(
        _bnadd_body,
        grid=(NBLK,),
        in_specs=[_row_spec(), _row_spec(), _full_spec((2, C)),
                  _full_spec((1, C)), _full_spec((1, C))],
        out_specs=[_row_spec()],
        out_shape=[jax.ShapeDtypeStruct((N, C), jnp.float32)],
        compiler_params=_SEQ,
    )(f, y, stats, g, b)[0]


def _sc_gather_max(x_hbm, gidx_hbm, out_hbm, x_sh, idx_v, rows_v, out_v,
                   gsem0, gsem1, osem0, osem1):
    sid = lax.axis_index("s")
    wid = sid * 2 + lax.axis_index("c")
    base = wid * RPW
    gsems = [gsem0, gsem1]
    osems = [osem0, osem1]

    @pl.when(sid < 15)
    def _():
        pltpu.sync_copy(x_hbm.at[pl.ds(sid * 624, 624)],
                        x_sh.at[pl.ds(sid * 624, 624)])

    @pl.when(sid == 15)
    def _():
        pltpu.sync_copy(x_hbm.at[pl.ds(15 * 624, N - 15 * 624)],
                        x_sh.at[pl.ds(15 * 624, N - 15 * 624)])

    nbatch = jnp.where(wid == NW - 1, (N - (NW - 1) * RPW) // NB, NBATCH)

    @pl.when(wid < NW - 1)
    def _():
        pltpu.sync_copy(gidx_hbm.at[pl.ds(base * K, RPW * K)], idx_v)

    @pl.when(wid == NW - 1)
    def _():
        tail = (N - (NW - 1) * RPW) * K
        pltpu.sync_copy(gidx_hbm.at[pl.ds((NW - 1) * RPW * K, tail)],
                        idx_v.at[pl.ds(0, tail)])

    plsc.subcore_barrier()

    def gcopy(b, s):
        return pltpu.make_async_copy(
            x_sh.at[idx_v.at[pl.ds(b * (NB * K), NB * K)]],
            rows_v.at[s], gsems[s])

    def ocopy(b, s):
        return pltpu.make_async_copy(
            out_v.at[s], out_hbm.at[pl.ds(base + b * NB, NB)], osems[s])

    def half(b, s):
        @pl.when(b + 1 < nbatch)
        def _():
            gcopy(b + 1, 1 - s).start()

        gcopy(b, s).wait()

        @pl.when(b >= 2)
        def _():
            ocopy(b - 2, s).wait()

        rv = rows_v.at[s]
        ov = out_v.at[s]

        def node_body(j, c):
            r0 = j * K
            for g in range(CP // 16):
                sl = pl.ds(g * 16, 16)
                w0 = rv[r0, sl]
                acc_hi = w0
                acc_lo = _shl16(w0)
                for r in range(1, K):
                    w = rv[r0 + r, sl]
                    acc_hi = jnp.maximum(acc_hi, w)
                    acc_lo = jnp.maximum(acc_lo, _shl16(w))
                hi_bits = _i32(acc_hi) & jnp.int32(-65536)
                lo_bits = lax.shift_right_logical(_i32(acc_lo), 16)
                ov[j, sl] = _f32(hi_bits | lo_bits)
            return c

        lax.fori_loop(0, NB, node_body, 0, unroll=2)
        ocopy(b, s).start()

    gcopy(0, 0).start()

    def outer(t, carry):
        half(t * 2, 0)
        half(t * 2 + 1, 1)
        return carry

    lax.fori_loop(0, nbatch // 2, outer, 0)
    ocopy(nbatch - 2, 0).wait()
    ocopy(nbatch - 1, 1).wait()


def _sc_pool_call(x, gidx_flat):
    mesh = plsc.VectorSubcoreMesh(core_axis_name="c", subcore_axis_name="s")
    kfn = functools.partial(
        pl.kernel,
        mesh=mesh,
        out_type=jax.ShapeDtypeStruct((N, CP), jnp.float32),
        scratch_types=[
            pltpu.VMEM_SHARED((N, CP), jnp.float32),
            pltpu.VMEM((RPW * K,), jnp.int32),
            pltpu.VMEM((2, NB * K, CP), jnp.float32),
            pltpu.VMEM((2, NB, CP), jnp.float32),
            pltpu.SemaphoreType.DMA,
            pltpu.SemaphoreType.DMA,
            pltpu.SemaphoreType.DMA,
            pltpu.SemaphoreType.DMA,
        ],
    )(_sc_gather_max)
    return kfn(x, gidx_flat)


def kernel(f, group_idx, params):
    mlp0 = params["mlp0"]
    blocks = params["blocks"]
    mlps = params["mlps"]

    gidx = group_idx.astype(jnp.int32).reshape(-1)

    def r1(a):
        return a.reshape(1, -1)

    y, s = _mlp_call(f, mlp0["W1"], r1(mlp0["b1"]), mlp0["W2"])
    cur_g, cur_b = r1(mlp0["g"]), r1(mlp0["b"])
    fcur = f
    packed = False

    for i in range(4):
        if packed:
            fcur, x = _bnmm_packed_call(fcur, y, cur_g, cur_b,
                                        blocks[i]["Wproj"])
        else:
            fcur, x = _bnmm_call(fcur, y, s, cur_g, cur_b,
                                 blocks[i]["Wproj"])
        y = _sc_pool_call(x, gidx)
        packed = True
        cur_g, cur_b = r1(blocks[i]["g"]), r1(blocks[i]["b"])
        if i % 2 == 1:
            m = mlps[i // 2]
            fcur, y, s = _bnmlp_call(fcur, y, cur_g, cur_b,
                                     m["W1"], r1(m["b1"]), m["W2"])
            packed = False
            cur_g, cur_b = r1(m["g"]), r1(m["b"])

    return _bnadd_call(fcur, y, s, cur_g, cur_b)
